# Initial kernel scaffold; baseline (speedup 1.0000x reference)
import jax
import jax.numpy as jnp
from jax import lax
from jax.experimental import pallas as pl
from jax.experimental.pallas import tpu as pltpu
from jax.experimental.pallas import tpu_sc as plsc

_B, _L, _D = 4096, 200, 32
_N = _B * _L
_G = _N - _L
_W = _L


def kernel(y, table):
    idx = y.reshape(1, _N).astype(jnp.int32)
    mesh = plsc.VectorSubcoreMesh(core_axis_name="c", subcore_axis_name="s")

    @pl.kernel(
        out_type=jax.ShapeDtypeStruct((_N, _D), jnp.float32),
        mesh=mesh,
        scratch_types=[
            pltpu.VMEM((_W, _D), jnp.float32),
            pltpu.SemaphoreType.DMA,
        ],
    )
    def _embed_shift(table_hbm, idx_hbm, out_hbm, zbuf, sem):
        first = jnp.logical_and(lax.axis_index("c") == 0,
                                lax.axis_index("s") == 0)

        @pl.when(first)
        def _zero_head():
            @pl.loop(0, _W)
            def _(r):
                @pl.loop(0, _D, step=16)
                def _(c0):
                    zbuf[r, pl.ds(c0, 16)] = jnp.zeros((16,), jnp.float32)
            pltpu.async_copy(zbuf, out_hbm.at[pl.ds(0, _W)], sem).wait()

        def body(i_vmem, o_vmem):
            pltpu.sync_copy(table_hbm.at[i_vmem.at[0]], o_vmem)

        pltpu.emit_pipeline(
            body,
            grid=(_G // _W,),
            in_specs=[pl.BlockSpec((1, _W), index_map=lambda i: (0, i))],
            out_specs=[pl.BlockSpec((_W, _D), index_map=lambda i: (i + 1, 0))],
            core_axis_name=("c", "s"),
            dimension_semantics=(pltpu.PARALLEL,),
        )(idx_hbm, out_hbm)

    out = _embed_shift(table, idx)
    return out.reshape(_B, _L, _D)


# speedup vs baseline: 1.4668x; 1.4668x over previous
import jax
import jax.numpy as jnp
from jax import lax
from jax.experimental import pallas as pl
from jax.experimental.pallas import tpu as pltpu
from jax.experimental.pallas import tpu_sc as plsc

_B, _L, _D = 4096, 200, 32
_N = _B * _L
_G = _N - _L
_W = _L


def kernel(y, table):
    idx = y.reshape(_N).astype(jnp.int32)
    mesh = plsc.VectorSubcoreMesh(core_axis_name="c", subcore_axis_name="s")

    @pl.kernel(
        out_type=jax.ShapeDtypeStruct((_N, _D), jnp.float32),
        mesh=mesh,
        scratch_types=[
            pltpu.VMEM((_W, _D), jnp.float32),
            pltpu.SemaphoreType.DMA,
        ],
        compiler_params=pltpu.CompilerParams(use_tc_tiling_on_sc=False),
    )
    def _embed_shift(table_hbm, idx_hbm, out_hbm, zbuf, sem):
        first = jnp.logical_and(lax.axis_index("c") == 0,
                                lax.axis_index("s") == 0)

        @pl.when(first)
        def _zero_head():
            @pl.loop(0, _W)
            def _(r):
                @pl.loop(0, _D, step=16)
                def _(c0):
                    zbuf[r, pl.ds(c0, 16)] = jnp.zeros((16,), jnp.float32)
            pltpu.async_copy(zbuf, out_hbm.at[pl.ds(0, _W)], sem).wait()

        def body(i_vmem, o_vmem):
            pltpu.sync_copy(table_hbm.at[i_vmem], o_vmem)

        pltpu.emit_pipeline(
            body,
            grid=(_G // _W,),
            in_specs=[pl.BlockSpec((_W,), index_map=lambda i: (i,))],
            out_specs=[pl.BlockSpec((_W, _D), index_map=lambda i: (i + 1, 0))],
            core_axis_name=("c", "s"),
            dimension_semantics=(pltpu.PARALLEL,),
        )(idx_hbm, out_hbm)

    out = _embed_shift(table, idx)
    return out.reshape(_B, _L, _D)


# speedup vs baseline: 1.4675x; 1.0005x over previous
import jax
import jax.numpy as jnp
from jax import lax
from jax.experimental import pallas as pl
from jax.experimental.pallas import tpu as pltpu
from jax.experimental.pallas import tpu_sc as plsc

_B, _L, _D = 4096, 200, 32
_N = _B * _L
_G = _N - _L
_W = _L


def kernel(y, table):
    idx = y.reshape(_N).astype(jnp.int32)
    mesh = plsc.VectorSubcoreMesh(core_axis_name="c", subcore_axis_name="s")

    @pl.kernel(
        out_type=jax.ShapeDtypeStruct((_B, _L, _D), jnp.float32),
        mesh=mesh,
        scratch_types=[
            pltpu.VMEM((1, _W, _D), jnp.float32),
            pltpu.SemaphoreType.DMA,
        ],
        compiler_params=pltpu.CompilerParams(use_tc_tiling_on_sc=False),
    )
    def _embed_shift(table_hbm, idx_hbm, out_hbm, zbuf, sem):
        first = jnp.logical_and(lax.axis_index("c") == 0,
                                lax.axis_index("s") == 0)

        @pl.when(first)
        def _zero_head():
            @pl.loop(0, _W)
            def _(r):
                @pl.loop(0, _D, step=16)
                def _(c0):
                    zbuf[0, r, pl.ds(c0, 16)] = jnp.zeros((16,), jnp.float32)
            pltpu.async_copy(zbuf, out_hbm.at[pl.ds(0, 1)], sem).wait()

        def body(i_vmem, o_vmem):
            pltpu.sync_copy(table_hbm.at[i_vmem], o_vmem.at[0])

        pltpu.emit_pipeline(
            body,
            grid=(_G // _W,),
            in_specs=[pl.BlockSpec((_W,), index_map=lambda i: (i,))],
            out_specs=[pl.BlockSpec((1, _W, _D),
                                    index_map=lambda i: (i + 1, 0, 0))],
            core_axis_name=("c", "s"),
            dimension_semantics=(pltpu.PARALLEL,),
        )(idx_hbm, out_hbm)

    return _embed_shift(table, idx)
